# baseline (speedup 1.0000x reference)
import functools

import jax
import jax.numpy as jnp
from jax import lax
from jax.experimental import pallas as pl
from jax.experimental.pallas import tpu as pltpu
from jax.experimental.pallas import tpu_sc as plsc

NUM_EXPERTS = 8
TOP_K = 2
HIDDEN = 1024
INTER = 768
SEQ = 2048

B = 256
NPAIR = SEQ * TOP_K
NP = NPAIR + NUM_EXPERTS * B
NB = NP // B

NC, NS, L = 2, 16, 16
NW = NC * NS
TPW = SEQ // NW
HCHUNK = 32
CW = 128

_sc_mesh = plsc.VectorSubcoreMesh(core_axis_name="c", subcore_axis_name="s")


@functools.partial(
    pl.kernel,
    mesh=_sc_mesh,
    out_type=(
        jax.ShapeDtypeStruct((NP, HIDDEN), jnp.float32),
        jax.ShapeDtypeStruct((NP, CW), jnp.float32),
    ),
    scratch_types=[
        pltpu.VMEM((TPW,), jnp.int32),
        pltpu.VMEM((TPW,), jnp.int32),
        pltpu.VMEM((TPW, HIDDEN), jnp.float32),
        pltpu.VMEM((TPW, CW), jnp.float32),
        pltpu.VMEM((TPW, CW), jnp.float32),
        pltpu.SemaphoreType.DMA,
        pltpu.SemaphoreType.DMA,
    ],
)
def _sc_scatter(x_hbm, d0_hbm, d1_hbm, tw0_hbm, tw1_hbm, xs_hbm, cs_hbm,
                idx0_v, idx1_v, rows_v, tw0_v, tw1_v, lsem, ssem):
    wid = lax.axis_index("s") * NC + lax.axis_index("c")
    base = wid * TPW
    loads = (
        pltpu.make_async_copy(d0_hbm.at[pl.ds(base, TPW)], idx0_v, lsem),
        pltpu.make_async_copy(d1_hbm.at[pl.ds(base, TPW)], idx1_v, lsem),
        pltpu.make_async_copy(x_hbm.at[pl.ds(base, TPW)], rows_v, lsem),
        pltpu.make_async_copy(tw0_hbm.at[pl.ds(base, TPW)], tw0_v, lsem),
        pltpu.make_async_copy(tw1_hbm.at[pl.ds(base, TPW)], tw1_v, lsem),
    )
    for c in loads:
        c.start()
    for c in loads:
        c.wait()
    stores = (
        pltpu.make_async_copy(rows_v, xs_hbm.at[idx0_v], ssem),
        pltpu.make_async_copy(rows_v, xs_hbm.at[idx1_v], ssem),
        pltpu.make_async_copy(tw0_v, cs_hbm.at[idx0_v], ssem),
        pltpu.make_async_copy(tw1_v, cs_hbm.at[idx1_v], ssem),
    )
    for c in stores:
        c.start()
    for c in stores:
        c.wait()


def _ffn_body(be_ref, x_ref, cs_ref, wg_ref, wu_ref, wd_ref, y_ref):
    @pl.when(be_ref[pl.program_id(0)] >= 0)
    def _():
        x = x_ref[...]
        g = lax.dot_general(x, wg_ref[0], (((1,), (1,)), ((), ())),
                            preferred_element_type=jnp.float32)
        u = lax.dot_general(x, wu_ref[0], (((1,), (1,)), ((), ())),
                            preferred_element_type=jnp.float32)
        h = g * lax.logistic(g) * u
        y = lax.dot_general(h, wd_ref[0], (((1,), (1,)), ((), ())),
                            preferred_element_type=jnp.float32)
        y_ref[...] = y * cs_ref[:, 0:1]


def _we(be, b):
    return jnp.maximum(be[b], 0)


def _ffn(block_expert, x_sorted, coef_sorted, w_gate, w_up, w_down):
    grid_spec = pltpu.PrefetchScalarGridSpec(
        num_scalar_prefetch=1,
        grid=(NB,),
        in_specs=[
            pl.BlockSpec((B, HIDDEN), lambda b, be: (b, 0)),
            pl.BlockSpec((B, CW), lambda b, be: (b, 0)),
            pl.BlockSpec((1, INTER, HIDDEN), lambda b, be: (_we(be, b), 0, 0)),
            pl.BlockSpec((1, INTER, HIDDEN), lambda b, be: (_we(be, b), 0, 0)),
            pl.BlockSpec((1, HIDDEN, INTER), lambda b, be: (_we(be, b), 0, 0)),
        ],
        out_specs=pl.BlockSpec((B, HIDDEN), lambda b, be: (b, 0)),
    )
    return pl.pallas_call(
        _ffn_body,
        grid_spec=grid_spec,
        out_shape=jax.ShapeDtypeStruct((NP, HIDDEN), jnp.float32),
        compiler_params=pltpu.CompilerParams(
            dimension_semantics=("parallel",),
        ),
    )(block_expert, x_sorted, coef_sorted, w_gate, w_up, w_down)


@functools.partial(
    pl.kernel,
    mesh=_sc_mesh,
    out_type=jax.ShapeDtypeStruct((SEQ, HIDDEN), jnp.float32),
    scratch_types=[
        pltpu.VMEM((TPW,), jnp.int32),
        pltpu.VMEM((TPW,), jnp.int32),
        pltpu.VMEM((HCHUNK, HIDDEN), jnp.float32),
        pltpu.VMEM((HCHUNK, HIDDEN), jnp.float32),
        pltpu.SemaphoreType.DMA,
        pltpu.SemaphoreType.DMA,
    ],
)
def _sc_combine(y_hbm, d0_hbm, d1_hbm, out_hbm,
                idx0_v, idx1_v, bufa, bufb, lsem, gsem):
    wid = lax.axis_index("s") * NC + lax.axis_index("c")
    base = wid * TPW
    l0 = pltpu.make_async_copy(d0_hbm.at[pl.ds(base, TPW)], idx0_v, lsem)
    l1 = pltpu.make_async_copy(d1_hbm.at[pl.ds(base, TPW)], idx1_v, lsem)
    l0.start()
    l1.start()
    l0.wait()
    l1.wait()

    for c in range(TPW // HCHUNK):
        lo = c * HCHUNK
        g0 = pltpu.make_async_copy(y_hbm.at[idx0_v.at[pl.ds(lo, HCHUNK)]],
                                   bufa, gsem)
        g1 = pltpu.make_async_copy(y_hbm.at[idx1_v.at[pl.ds(lo, HCHUNK)]],
                                   bufb, gsem)
        g0.start()
        g1.start()
        g0.wait()
        g1.wait()

        def row_body(i, carry):
            for j in range(HIDDEN // L):
                bufa[i, pl.ds(j * L, L)] = (bufa[i, pl.ds(j * L, L)]
                                            + bufb[i, pl.ds(j * L, L)])
            return carry

        lax.fori_loop(0, HCHUNK, row_body, 0)
        pltpu.sync_copy(bufa, out_hbm.at[pl.ds(base + lo, HCHUNK)])


def kernel(hidden_states, top_k_index, top_k_weights, w_gate, w_up, w_down):
    orig_shape = hidden_states.shape
    x = hidden_states.reshape(SEQ, HIDDEN)
    ids = top_k_index.reshape(NPAIR)
    tw = top_k_weights.reshape(SEQ, TOP_K)

    onehot = (ids[:, None] == jnp.arange(NUM_EXPERTS, dtype=jnp.int32))
    oh32 = onehot.astype(jnp.int32)
    counts = jnp.sum(oh32, axis=0)
    rank = jnp.cumsum(oh32, axis=0) - 1
    padded = ((counts + B - 1) // B) * B
    pstart = jnp.concatenate(
        [jnp.zeros((1,), jnp.int32), jnp.cumsum(padded)[:-1].astype(jnp.int32)])
    dest = jnp.sum(oh32 * (pstart[None, :] + rank), axis=1)
    d = dest.reshape(SEQ, TOP_K)
    d0, d1 = d[:, 0], d[:, 1]
    start_blk = pstart // B
    nb_used = (pstart[-1] + padded[-1]) // B
    biota = jnp.arange(NB, dtype=jnp.int32)
    block_expert = (jnp.sum(
        (biota[:, None] >= start_blk[None, :])
        .astype(jnp.int32), axis=1) - 1).astype(jnp.int32)
    block_expert = jnp.where(biota < nb_used, block_expert, -1)

    tw0b = jnp.broadcast_to(tw[:, 0:1], (SEQ, CW))
    tw1b = jnp.broadcast_to(tw[:, 1:2], (SEQ, CW))

    x_sorted, coef_sorted = _sc_scatter(x, d0, d1, tw0b, tw1b)
    y_sorted = _ffn(block_expert, x_sorted, coef_sorted, w_gate, w_up, w_down)
    out = _sc_combine(y_sorted, d0, d1)
    return out.reshape(orig_shape)


# speedup vs baseline: 1.0387x; 1.0387x over previous
import functools

import jax
import jax.numpy as jnp
from jax import lax
from jax.experimental import pallas as pl
from jax.experimental.pallas import tpu as pltpu
from jax.experimental.pallas import tpu_sc as plsc

NUM_EXPERTS = 8
TOP_K = 2
HIDDEN = 1024
INTER = 768
SEQ = 2048

B = 512
NPAIR = SEQ * TOP_K
NP = NPAIR + NUM_EXPERTS * B
NB = NP // B

NC, NS, L = 2, 16, 16
NW = NC * NS
TPW = SEQ // NW
HCHUNK = 16
CW = 128

_sc_mesh = plsc.VectorSubcoreMesh(core_axis_name="c", subcore_axis_name="s")


@functools.partial(
    pl.kernel,
    mesh=_sc_mesh,
    out_type=(
        jax.ShapeDtypeStruct((NP, HIDDEN), jnp.float32),
        jax.ShapeDtypeStruct((NP, CW), jnp.float32),
    ),
    scratch_types=[
        pltpu.VMEM((TPW,), jnp.int32),
        pltpu.VMEM((TPW,), jnp.int32),
        pltpu.VMEM((TPW, HIDDEN), jnp.float32),
        pltpu.VMEM((TPW, CW), jnp.float32),
        pltpu.VMEM((TPW, CW), jnp.float32),
        pltpu.SemaphoreType.DMA,
        pltpu.SemaphoreType.DMA,
    ],
)
def _sc_scatter(x_hbm, d0_hbm, d1_hbm, tw0_hbm, tw1_hbm, xs_hbm, cs_hbm,
                idx0_v, idx1_v, rows_v, tw0_v, tw1_v, lsem, ssem):
    wid = lax.axis_index("s") * NC + lax.axis_index("c")
    base = wid * TPW
    loads = (
        pltpu.make_async_copy(d0_hbm.at[pl.ds(base, TPW)], idx0_v, lsem),
        pltpu.make_async_copy(d1_hbm.at[pl.ds(base, TPW)], idx1_v, lsem),
        pltpu.make_async_copy(x_hbm.at[pl.ds(base, TPW)], rows_v, lsem),
        pltpu.make_async_copy(tw0_hbm.at[pl.ds(base, TPW)], tw0_v, lsem),
        pltpu.make_async_copy(tw1_hbm.at[pl.ds(base, TPW)], tw1_v, lsem),
    )
    for c in loads:
        c.start()
    for c in loads:
        c.wait()
    stores = (
        pltpu.make_async_copy(rows_v, xs_hbm.at[idx0_v], ssem),
        pltpu.make_async_copy(rows_v, xs_hbm.at[idx1_v], ssem),
        pltpu.make_async_copy(tw0_v, cs_hbm.at[idx0_v], ssem),
        pltpu.make_async_copy(tw1_v, cs_hbm.at[idx1_v], ssem),
    )
    for c in stores:
        c.start()
    for c in stores:
        c.wait()


def _ffn_body(be_ref, x_ref, cs_ref, wg_ref, wu_ref, wd_ref, y_ref):
    @pl.when(be_ref[pl.program_id(0)] >= 0)
    def _():
        x = x_ref[...]
        g = lax.dot_general(x, wg_ref[0], (((1,), (1,)), ((), ())),
                            preferred_element_type=jnp.float32)
        u = lax.dot_general(x, wu_ref[0], (((1,), (1,)), ((), ())),
                            preferred_element_type=jnp.float32)
        h = g * lax.logistic(g) * u
        y = lax.dot_general(h, wd_ref[0], (((1,), (1,)), ((), ())),
                            preferred_element_type=jnp.float32)
        y_ref[...] = y * cs_ref[:, 0:1]


def _we(be, b):
    return jnp.maximum(be[b], 0)


def _ffn(block_expert, x_sorted, coef_sorted, w_gate, w_up, w_down):
    grid_spec = pltpu.PrefetchScalarGridSpec(
        num_scalar_prefetch=1,
        grid=(NB,),
        in_specs=[
            pl.BlockSpec((B, HIDDEN), lambda b, be: (b, 0)),
            pl.BlockSpec((B, CW), lambda b, be: (b, 0)),
            pl.BlockSpec((1, INTER, HIDDEN), lambda b, be: (_we(be, b), 0, 0)),
            pl.BlockSpec((1, INTER, HIDDEN), lambda b, be: (_we(be, b), 0, 0)),
            pl.BlockSpec((1, HIDDEN, INTER), lambda b, be: (_we(be, b), 0, 0)),
        ],
        out_specs=pl.BlockSpec((B, HIDDEN), lambda b, be: (b, 0)),
    )
    return pl.pallas_call(
        _ffn_body,
        grid_spec=grid_spec,
        out_shape=jax.ShapeDtypeStruct((NP, HIDDEN), jnp.float32),
        compiler_params=pltpu.CompilerParams(
            dimension_semantics=("parallel",),
        ),
    )(block_expert, x_sorted, coef_sorted, w_gate, w_up, w_down)


@functools.partial(
    pl.kernel,
    mesh=_sc_mesh,
    out_type=jax.ShapeDtypeStruct((SEQ, HIDDEN), jnp.float32),
    scratch_types=[
        pltpu.VMEM((TPW,), jnp.int32),
        pltpu.VMEM((TPW,), jnp.int32),
        pltpu.VMEM((2, HCHUNK, HIDDEN), jnp.float32),
        pltpu.VMEM((2, HCHUNK, HIDDEN), jnp.float32),
        pltpu.SemaphoreType.DMA,
        pltpu.SemaphoreType.DMA,
    ],
)
def _sc_combine(y_hbm, d0_hbm, d1_hbm, out_hbm,
                idx0_v, idx1_v, bufa, bufb, lsem, gsem):
    wid = lax.axis_index("s") * NC + lax.axis_index("c")
    base = wid * TPW
    l0 = pltpu.make_async_copy(d0_hbm.at[pl.ds(base, TPW)], idx0_v, lsem)
    l1 = pltpu.make_async_copy(d1_hbm.at[pl.ds(base, TPW)], idx1_v, lsem)
    l0.start()
    l1.start()
    l0.wait()
    l1.wait()

    nchunk = TPW // HCHUNK

    def gathers(c):
        lo = c * HCHUNK
        s = c % 2
        g0 = pltpu.make_async_copy(y_hbm.at[idx0_v.at[pl.ds(lo, HCHUNK)]],
                                   bufa.at[s], gsem)
        g1 = pltpu.make_async_copy(y_hbm.at[idx1_v.at[pl.ds(lo, HCHUNK)]],
                                   bufb.at[s], gsem)
        return g0, g1

    for g in gathers(0):
        g.start()
    for c in range(nchunk):
        s = c % 2
        for g in gathers(c):
            g.wait()
        if c + 1 < nchunk:
            for g in gathers(c + 1):
                g.start()

        def row_body(i, carry):
            for j in range(HIDDEN // L):
                bufa[s, i, pl.ds(j * L, L)] = (bufa[s, i, pl.ds(j * L, L)]
                                               + bufb[s, i, pl.ds(j * L, L)])
            return carry

        lax.fori_loop(0, HCHUNK, row_body, 0)
        pltpu.sync_copy(bufa.at[s], out_hbm.at[pl.ds(base + c * HCHUNK, HCHUNK)])


def kernel(hidden_states, top_k_index, top_k_weights, w_gate, w_up, w_down):
    orig_shape = hidden_states.shape
    x = hidden_states.reshape(SEQ, HIDDEN)
    ids = top_k_index.reshape(NPAIR)
    tw = top_k_weights.reshape(SEQ, TOP_K)

    onehot = (ids[:, None] == jnp.arange(NUM_EXPERTS, dtype=jnp.int32))
    oh32 = onehot.astype(jnp.int32)
    counts = jnp.sum(oh32, axis=0)
    rank = jnp.cumsum(oh32, axis=0) - 1
    padded = ((counts + B - 1) // B) * B
    pstart = jnp.concatenate(
        [jnp.zeros((1,), jnp.int32), jnp.cumsum(padded)[:-1].astype(jnp.int32)])
    dest = jnp.sum(oh32 * (pstart[None, :] + rank), axis=1)
    d = dest.reshape(SEQ, TOP_K)
    d0, d1 = d[:, 0], d[:, 1]
    start_blk = pstart // B
    nb_used = (pstart[-1] + padded[-1]) // B
    biota = jnp.arange(NB, dtype=jnp.int32)
    block_expert = (jnp.sum(
        (biota[:, None] >= start_blk[None, :])
        .astype(jnp.int32), axis=1) - 1).astype(jnp.int32)
    block_expert = jnp.where(biota < nb_used, block_expert, -1)

    tw0b = jnp.broadcast_to(tw[:, 0:1], (SEQ, CW))
    tw1b = jnp.broadcast_to(tw[:, 1:2], (SEQ, CW))

    x_sorted, coef_sorted = _sc_scatter(x, d0, d1, tw0b, tw1b)
    y_sorted = _ffn(block_expert, x_sorted, coef_sorted, w_gate, w_up, w_down)
    out = _sc_combine(y_sorted, d0, d1)
    return out.reshape(orig_shape)


# speedup vs baseline: 1.0908x; 1.0501x over previous
import functools

import jax
import jax.numpy as jnp
from jax import lax
from jax.experimental import pallas as pl
from jax.experimental.pallas import tpu as pltpu
from jax.experimental.pallas import tpu_sc as plsc

NUM_EXPERTS = 8
TOP_K = 2
HIDDEN = 1024
INTER = 768
SEQ = 2048

B = 512
NPAIR = SEQ * TOP_K
NP = NPAIR + NUM_EXPERTS * B
NB = NP // B

NC, NS, L = 2, 16, 16
NW = NC * NS
TPW = SEQ // NW
HCHUNK = 16
CW = 128

_sc_mesh = plsc.VectorSubcoreMesh(core_axis_name="c", subcore_axis_name="s")


def _meta_body(idsT_ref, tw_ref, d0_ref, d1_ref, be_ref, tw0b_ref, tw1b_ref):
    idsT = idsT_ref[...]
    eiota = lax.broadcasted_iota(jnp.int32, (NUM_EXPERTS, SEQ), 0)
    c0 = (jnp.broadcast_to(idsT[0:1, :], (NUM_EXPERTS, SEQ)) == eiota
          ).astype(jnp.int32)
    c1 = (jnp.broadcast_to(idsT[1:2, :], (NUM_EXPERTS, SEQ)) == eiota
          ).astype(jnp.int32)
    s = c0 + c1
    cs = s
    k = 1
    laneq = lax.broadcasted_iota(jnp.int32, (NUM_EXPERTS, SEQ), 1)
    while k < SEQ:
        shifted = jnp.concatenate(
            [jnp.zeros((NUM_EXPERTS, k), jnp.int32), cs[:, :SEQ - k]], axis=1)
        cs = cs + shifted
        k *= 2
    excl = cs - s
    counts = cs[:, SEQ - 1:SEQ]
    padded = ((counts + B - 1) // B) * B
    ps = padded
    k = 1
    while k < NUM_EXPERTS:
        ps = ps + jnp.concatenate(
            [jnp.zeros((k, 1), jnp.int32), ps[:NUM_EXPERTS - k, :]], axis=0)
        k *= 2
    pstart = ps - padded
    d0_ref[...] = jnp.sum(c0 * (pstart + excl), axis=0, keepdims=True)
    d1_ref[...] = jnp.sum(c1 * (pstart + excl + c0), axis=0, keepdims=True)
    start_blk = pstart // B
    nb_used = jnp.sum(padded) // B
    biota = lax.broadcasted_iota(jnp.int32, (1, NB), 1)
    be = jnp.sum((jnp.broadcast_to(biota, (NUM_EXPERTS, NB))
                  >= start_blk).astype(jnp.int32), axis=0, keepdims=True) - 1
    be_ref[...] = jnp.where(biota < nb_used, be, -1)
    tw = tw_ref[...]
    tw0b_ref[...] = jnp.broadcast_to(tw[:, 0:1], (SEQ, CW))
    tw1b_ref[...] = jnp.broadcast_to(tw[:, 1:2], (SEQ, CW))


def _meta(idsT, tw):
    return pl.pallas_call(
        _meta_body,
        grid=(1,),
        in_specs=[
            pl.BlockSpec((TOP_K, SEQ), lambda i: (0, 0)),
            pl.BlockSpec((SEQ, TOP_K), lambda i: (0, 0)),
        ],
        out_specs=[
            pl.BlockSpec((1, SEQ), lambda i: (0, 0)),
            pl.BlockSpec((1, SEQ), lambda i: (0, 0)),
            pl.BlockSpec((1, NB), lambda i: (0, 0)),
            pl.BlockSpec((SEQ, CW), lambda i: (0, 0)),
            pl.BlockSpec((SEQ, CW), lambda i: (0, 0)),
        ],
        out_shape=[
            jax.ShapeDtypeStruct((1, SEQ), jnp.int32),
            jax.ShapeDtypeStruct((1, SEQ), jnp.int32),
            jax.ShapeDtypeStruct((1, NB), jnp.int32),
            jax.ShapeDtypeStruct((SEQ, CW), jnp.float32),
            jax.ShapeDtypeStruct((SEQ, CW), jnp.float32),
        ],
    )(idsT, tw)


@functools.partial(
    pl.kernel,
    mesh=_sc_mesh,
    out_type=(
        jax.ShapeDtypeStruct((NP, HIDDEN), jnp.float32),
        jax.ShapeDtypeStruct((NP, CW), jnp.float32),
    ),
    scratch_types=[
        pltpu.VMEM((TPW,), jnp.int32),
        pltpu.VMEM((TPW,), jnp.int32),
        pltpu.VMEM((TPW, HIDDEN), jnp.float32),
        pltpu.VMEM((TPW, CW), jnp.float32),
        pltpu.VMEM((TPW, CW), jnp.float32),
        pltpu.SemaphoreType.DMA,
        pltpu.SemaphoreType.DMA,
    ],
)
def _sc_scatter(x_hbm, d0_hbm, d1_hbm, tw0_hbm, tw1_hbm, xs_hbm, cs_hbm,
                idx0_v, idx1_v, rows_v, tw0_v, tw1_v, lsem, ssem):
    wid = lax.axis_index("s") * NC + lax.axis_index("c")
    base = wid * TPW
    loads = (
        pltpu.make_async_copy(d0_hbm.at[pl.ds(base, TPW)], idx0_v, lsem),
        pltpu.make_async_copy(d1_hbm.at[pl.ds(base, TPW)], idx1_v, lsem),
        pltpu.make_async_copy(x_hbm.at[pl.ds(base, TPW)], rows_v, lsem),
        pltpu.make_async_copy(tw0_hbm.at[pl.ds(base, TPW)], tw0_v, lsem),
        pltpu.make_async_copy(tw1_hbm.at[pl.ds(base, TPW)], tw1_v, lsem),
    )
    for c in loads:
        c.start()
    for c in loads:
        c.wait()
    stores = (
        pltpu.make_async_copy(rows_v, xs_hbm.at[idx0_v], ssem),
        pltpu.make_async_copy(rows_v, xs_hbm.at[idx1_v], ssem),
        pltpu.make_async_copy(tw0_v, cs_hbm.at[idx0_v], ssem),
        pltpu.make_async_copy(tw1_v, cs_hbm.at[idx1_v], ssem),
    )
    for c in stores:
        c.start()
    for c in stores:
        c.wait()


def _ffn_body(be_ref, x_ref, cs_ref, wg_ref, wu_ref, wd_ref, y_ref):
    @pl.when(be_ref[pl.program_id(0)] >= 0)
    def _():
        x = x_ref[...]
        g = lax.dot_general(x, wg_ref[0], (((1,), (1,)), ((), ())),
                            preferred_element_type=jnp.float32)
        u = lax.dot_general(x, wu_ref[0], (((1,), (1,)), ((), ())),
                            preferred_element_type=jnp.float32)
        h = g * lax.logistic(g) * u
        y = lax.dot_general(h, wd_ref[0], (((1,), (1,)), ((), ())),
                            preferred_element_type=jnp.float32)
        y_ref[...] = y * cs_ref[:, 0:1]


def _we(be, b):
    return jnp.maximum(be[b], 0)


def _ffn(block_expert, x_sorted, coef_sorted, w_gate, w_up, w_down):
    grid_spec = pltpu.PrefetchScalarGridSpec(
        num_scalar_prefetch=1,
        grid=(NB,),
        in_specs=[
            pl.BlockSpec((B, HIDDEN), lambda b, be: (b, 0)),
            pl.BlockSpec((B, CW), lambda b, be: (b, 0)),
            pl.BlockSpec((1, INTER, HIDDEN), lambda b, be: (_we(be, b), 0, 0)),
            pl.BlockSpec((1, INTER, HIDDEN), lambda b, be: (_we(be, b), 0, 0)),
            pl.BlockSpec((1, HIDDEN, INTER), lambda b, be: (_we(be, b), 0, 0)),
        ],
        out_specs=pl.BlockSpec((B, HIDDEN), lambda b, be: (b, 0)),
    )
    return pl.pallas_call(
        _ffn_body,
        grid_spec=grid_spec,
        out_shape=jax.ShapeDtypeStruct((NP, HIDDEN), jnp.float32),
        compiler_params=pltpu.CompilerParams(
            dimension_semantics=("parallel",),
        ),
    )(block_expert, x_sorted, coef_sorted, w_gate, w_up, w_down)


@functools.partial(
    pl.kernel,
    mesh=_sc_mesh,
    out_type=jax.ShapeDtypeStruct((SEQ, HIDDEN), jnp.float32),
    scratch_types=[
        pltpu.VMEM((TPW,), jnp.int32),
        pltpu.VMEM((TPW,), jnp.int32),
        pltpu.VMEM((2, HCHUNK, HIDDEN), jnp.float32),
        pltpu.VMEM((2, HCHUNK, HIDDEN), jnp.float32),
        pltpu.SemaphoreType.DMA,
        pltpu.SemaphoreType.DMA,
    ],
)
def _sc_combine(y_hbm, d0_hbm, d1_hbm, out_hbm,
                idx0_v, idx1_v, bufa, bufb, lsem, gsem):
    wid = lax.axis_index("s") * NC + lax.axis_index("c")
    base = wid * TPW
    l0 = pltpu.make_async_copy(d0_hbm.at[pl.ds(base, TPW)], idx0_v, lsem)
    l1 = pltpu.make_async_copy(d1_hbm.at[pl.ds(base, TPW)], idx1_v, lsem)
    l0.start()
    l1.start()
    l0.wait()
    l1.wait()

    nchunk = TPW // HCHUNK

    def gathers(c):
        lo = c * HCHUNK
        s = c % 2
        g0 = pltpu.make_async_copy(y_hbm.at[idx0_v.at[pl.ds(lo, HCHUNK)]],
                                   bufa.at[s], gsem)
        g1 = pltpu.make_async_copy(y_hbm.at[idx1_v.at[pl.ds(lo, HCHUNK)]],
                                   bufb.at[s], gsem)
        return g0, g1

    for g in gathers(0):
        g.start()
    for c in range(nchunk):
        s = c % 2
        for g in gathers(c):
            g.wait()
        if c + 1 < nchunk:
            for g in gathers(c + 1):
                g.start()

        def row_body(i, carry):
            for j in range(HIDDEN // L):
                bufa[s, i, pl.ds(j * L, L)] = (bufa[s, i, pl.ds(j * L, L)]
                                               + bufb[s, i, pl.ds(j * L, L)])
            return carry

        lax.fori_loop(0, HCHUNK, row_body, 0)
        pltpu.sync_copy(bufa.at[s], out_hbm.at[pl.ds(base + c * HCHUNK, HCHUNK)])


def kernel(hidden_states, top_k_index, top_k_weights, w_gate, w_up, w_down):
    orig_shape = hidden_states.shape
    x = hidden_states.reshape(SEQ, HIDDEN)
    ids = top_k_index.reshape(NPAIR)
    tw = top_k_weights.reshape(SEQ, TOP_K)

    d0_2d, d1_2d, be_2d, tw0b, tw1b = _meta(ids.reshape(SEQ, TOP_K).T, tw)
    d0 = d0_2d.reshape(SEQ)
    d1 = d1_2d.reshape(SEQ)
    block_expert = be_2d.reshape(NB)

    x_sorted, coef_sorted = _sc_scatter(x, d0, d1, tw0b, tw1b)
    y_sorted = _ffn(block_expert, x_sorted, coef_sorted, w_gate, w_up, w_down)
    out = _sc_combine(y_sorted, d0, d1)
    return out.reshape(orig_shape)


# speedup vs baseline: 1.1212x; 1.0279x over previous
import functools

import jax
import jax.numpy as jnp
from jax import lax
from jax.experimental import pallas as pl
from jax.experimental.pallas import tpu as pltpu
from jax.experimental.pallas import tpu_sc as plsc

NUM_EXPERTS = 8
TOP_K = 2
HIDDEN = 1024
INTER = 768
SEQ = 2048

B = 512
NPAIR = SEQ * TOP_K
NP = NPAIR + NUM_EXPERTS * B
NB = NP // B

NC, NS, L = 2, 16, 16
NW = NC * NS
TPW = SEQ // NW
HCHUNK = 16
CW = 128

_sc_mesh = plsc.VectorSubcoreMesh(core_axis_name="c", subcore_axis_name="s")


def _meta_body(idsT_ref, tw_ref, d0_ref, d1_ref, be_ref, tw0b_ref, tw1b_ref):
    idsT = idsT_ref[...]
    eiota = lax.broadcasted_iota(jnp.int32, (NUM_EXPERTS, SEQ), 0)
    c0 = (jnp.broadcast_to(idsT[0:1, :], (NUM_EXPERTS, SEQ)) == eiota
          ).astype(jnp.int32)
    c1 = (jnp.broadcast_to(idsT[1:2, :], (NUM_EXPERTS, SEQ)) == eiota
          ).astype(jnp.int32)
    s = c0 + c1
    cs = s
    k = 1
    laneq = lax.broadcasted_iota(jnp.int32, (NUM_EXPERTS, SEQ), 1)
    while k < SEQ:
        shifted = jnp.concatenate(
            [jnp.zeros((NUM_EXPERTS, k), jnp.int32), cs[:, :SEQ - k]], axis=1)
        cs = cs + shifted
        k *= 2
    excl = cs - s
    counts = cs[:, SEQ - 1:SEQ]
    padded = ((counts + B - 1) // B) * B
    ps = padded
    k = 1
    while k < NUM_EXPERTS:
        ps = ps + jnp.concatenate(
            [jnp.zeros((k, 1), jnp.int32), ps[:NUM_EXPERTS - k, :]], axis=0)
        k *= 2
    pstart = ps - padded
    d0_ref[...] = jnp.sum(c0 * (pstart + excl), axis=0, keepdims=True)
    d1_ref[...] = jnp.sum(c1 * (pstart + excl + c0), axis=0, keepdims=True)
    start_blk = pstart // B
    nb_used = jnp.sum(padded) // B
    biota = lax.broadcasted_iota(jnp.int32, (1, NB), 1)
    be = jnp.sum((jnp.broadcast_to(biota, (NUM_EXPERTS, NB))
                  >= start_blk).astype(jnp.int32), axis=0, keepdims=True) - 1
    be_ref[...] = jnp.where(biota < nb_used, be, -1)
    tw = tw_ref[...]
    tw0b_ref[...] = jnp.broadcast_to(tw[:, 0:1], (SEQ, CW))
    tw1b_ref[...] = jnp.broadcast_to(tw[:, 1:2], (SEQ, CW))


def _meta(idsT, tw):
    return pl.pallas_call(
        _meta_body,
        grid=(1,),
        in_specs=[
            pl.BlockSpec((TOP_K, SEQ), lambda i: (0, 0)),
            pl.BlockSpec((SEQ, TOP_K), lambda i: (0, 0)),
        ],
        out_specs=[
            pl.BlockSpec((1, SEQ), lambda i: (0, 0)),
            pl.BlockSpec((1, SEQ), lambda i: (0, 0)),
            pl.BlockSpec((1, NB), lambda i: (0, 0)),
            pl.BlockSpec((SEQ, CW), lambda i: (0, 0)),
            pl.BlockSpec((SEQ, CW), lambda i: (0, 0)),
        ],
        out_shape=[
            jax.ShapeDtypeStruct((1, SEQ), jnp.int32),
            jax.ShapeDtypeStruct((1, SEQ), jnp.int32),
            jax.ShapeDtypeStruct((1, NB), jnp.int32),
            jax.ShapeDtypeStruct((SEQ, CW), jnp.float32),
            jax.ShapeDtypeStruct((SEQ, CW), jnp.float32),
        ],
    )(idsT, tw)


@functools.partial(
    pl.kernel,
    mesh=_sc_mesh,
    out_type=(
        jax.ShapeDtypeStruct((NP, HIDDEN), jnp.float32),
        jax.ShapeDtypeStruct((NP, CW), jnp.float32),
    ),
    scratch_types=[
        pltpu.VMEM((TPW,), jnp.int32),
        pltpu.VMEM((TPW,), jnp.int32),
        pltpu.VMEM((TPW, HIDDEN), jnp.float32),
        pltpu.VMEM((TPW, CW), jnp.float32),
        pltpu.VMEM((TPW, CW), jnp.float32),
        pltpu.SemaphoreType.DMA,
        pltpu.SemaphoreType.DMA,
    ],
)
def _sc_scatter(x_hbm, d0_hbm, d1_hbm, tw0_hbm, tw1_hbm, xs_hbm, cs_hbm,
                idx0_v, idx1_v, rows_v, tw0_v, tw1_v, lsem, ssem):
    wid = lax.axis_index("s") * NC + lax.axis_index("c")
    base = wid * TPW
    loads = (
        pltpu.make_async_copy(d0_hbm.at[pl.ds(base, TPW)], idx0_v, lsem),
        pltpu.make_async_copy(d1_hbm.at[pl.ds(base, TPW)], idx1_v, lsem),
        pltpu.make_async_copy(x_hbm.at[pl.ds(base, TPW)], rows_v, lsem),
        pltpu.make_async_copy(tw0_hbm.at[pl.ds(base, TPW)], tw0_v, lsem),
        pltpu.make_async_copy(tw1_hbm.at[pl.ds(base, TPW)], tw1_v, lsem),
    )
    for c in loads:
        c.start()
    for c in loads:
        c.wait()
    stores = (
        pltpu.make_async_copy(rows_v, xs_hbm.at[idx0_v], ssem),
        pltpu.make_async_copy(rows_v, xs_hbm.at[idx1_v], ssem),
        pltpu.make_async_copy(tw0_v, cs_hbm.at[idx0_v], ssem),
        pltpu.make_async_copy(tw1_v, cs_hbm.at[idx1_v], ssem),
    )
    for c in stores:
        c.start()
    for c in stores:
        c.wait()


def _ffn_body(be_ref, x_ref, cs_ref, wg_ref, wu_ref, wd_ref, y_ref):
    @pl.when(be_ref[pl.program_id(0)] >= 0)
    def _():
        x = x_ref[...]
        g = lax.dot_general(x, wg_ref[0], (((1,), (1,)), ((), ())),
                            preferred_element_type=jnp.float32)
        u = lax.dot_general(x, wu_ref[0], (((1,), (1,)), ((), ())),
                            preferred_element_type=jnp.float32)
        h = g * lax.logistic(g) * u
        y = lax.dot_general(h, wd_ref[0], (((1,), (1,)), ((), ())),
                            preferred_element_type=jnp.float32)
        y_ref[...] = y * cs_ref[:, 0:1]


def _we(be, b):
    return jnp.maximum(be[b], 0)


def _live(be, b, alt):
    return jnp.where(be[b] >= 0, b, alt)


def _ffn(block_expert, x_sorted, coef_sorted, w_gate, w_up, w_down):
    grid_spec = pltpu.PrefetchScalarGridSpec(
        num_scalar_prefetch=1,
        grid=(NB,),
        in_specs=[
            pl.BlockSpec((B, HIDDEN), lambda b, be: (_live(be, b, 0), 0)),
            pl.BlockSpec((B, CW), lambda b, be: (_live(be, b, 0), 0)),
            pl.BlockSpec((1, INTER, HIDDEN), lambda b, be: (_we(be, b), 0, 0)),
            pl.BlockSpec((1, INTER, HIDDEN), lambda b, be: (_we(be, b), 0, 0)),
            pl.BlockSpec((1, HIDDEN, INTER), lambda b, be: (_we(be, b), 0, 0)),
        ],
        out_specs=pl.BlockSpec((B, HIDDEN),
                               lambda b, be: (_live(be, b, NB - 1), 0)),
    )
    return pl.pallas_call(
        _ffn_body,
        grid_spec=grid_spec,
        out_shape=jax.ShapeDtypeStruct((NP, HIDDEN), jnp.float32),
        compiler_params=pltpu.CompilerParams(
            dimension_semantics=("parallel",),
        ),
    )(block_expert, x_sorted, coef_sorted, w_gate, w_up, w_down)


@functools.partial(
    pl.kernel,
    mesh=_sc_mesh,
    out_type=jax.ShapeDtypeStruct((SEQ, HIDDEN), jnp.float32),
    scratch_types=[
        pltpu.VMEM((TPW,), jnp.int32),
        pltpu.VMEM((TPW,), jnp.int32),
        pltpu.VMEM((2, HCHUNK, HIDDEN), jnp.float32),
        pltpu.VMEM((2, HCHUNK, HIDDEN), jnp.float32),
        pltpu.SemaphoreType.DMA,
        pltpu.SemaphoreType.DMA,
    ],
)
def _sc_combine(y_hbm, d0_hbm, d1_hbm, out_hbm,
                idx0_v, idx1_v, bufa, bufb, lsem, gsem):
    wid = lax.axis_index("s") * NC + lax.axis_index("c")
    base = wid * TPW
    l0 = pltpu.make_async_copy(d0_hbm.at[pl.ds(base, TPW)], idx0_v, lsem)
    l1 = pltpu.make_async_copy(d1_hbm.at[pl.ds(base, TPW)], idx1_v, lsem)
    l0.start()
    l1.start()
    l0.wait()
    l1.wait()

    nchunk = TPW // HCHUNK

    def gathers(c):
        lo = c * HCHUNK
        s = c % 2
        g0 = pltpu.make_async_copy(y_hbm.at[idx0_v.at[pl.ds(lo, HCHUNK)]],
                                   bufa.at[s], gsem)
        g1 = pltpu.make_async_copy(y_hbm.at[idx1_v.at[pl.ds(lo, HCHUNK)]],
                                   bufb.at[s], gsem)
        return g0, g1

    for g in gathers(0):
        g.start()
    for c in range(nchunk):
        s = c % 2
        for g in gathers(c):
            g.wait()
        if c + 1 < nchunk:
            for g in gathers(c + 1):
                g.start()

        def row_body(i, carry):
            for j in range(HIDDEN // L):
                bufa[s, i, pl.ds(j * L, L)] = (bufa[s, i, pl.ds(j * L, L)]
                                               + bufb[s, i, pl.ds(j * L, L)])
            return carry

        lax.fori_loop(0, HCHUNK, row_body, 0)
        pltpu.sync_copy(bufa.at[s], out_hbm.at[pl.ds(base + c * HCHUNK, HCHUNK)])


def kernel(hidden_states, top_k_index, top_k_weights, w_gate, w_up, w_down):
    orig_shape = hidden_states.shape
    x = hidden_states.reshape(SEQ, HIDDEN)
    ids = top_k_index.reshape(NPAIR)
    tw = top_k_weights.reshape(SEQ, TOP_K)

    d0_2d, d1_2d, be_2d, tw0b, tw1b = _meta(ids.reshape(SEQ, TOP_K).T, tw)
    d0 = d0_2d.reshape(SEQ)
    d1 = d1_2d.reshape(SEQ)
    block_expert = be_2d.reshape(NB)

    x_sorted, coef_sorted = _sc_scatter(x, d0, d1, tw0b, tw1b)
    y_sorted = _ffn(block_expert, x_sorted, coef_sorted, w_gate, w_up, w_down)
    out = _sc_combine(y_sorted, d0, d1)
    return out.reshape(orig_shape)
